# baseline (speedup 1.0000x reference)
import functools

import jax
import jax.numpy as jnp
from jax import lax
from jax.experimental import pallas as pl
from jax.experimental.pallas import tpu as pltpu
from jax.experimental.pallas import tpu_sc as plsc

jax.config.update("jax_enable_x64", True)

_FACTOR = 13.605693122994 * 0.5291772105638411

_N = 100000
_M = 64
_NW = 32
_BLK = 128
_HJ = _M // 2
_L = 16
_GRP = _BLK // _L
_BLKS_MAIN = 25
_BLKS_LAST = 6
_TAIL0 = 31 * _BLKS_MAIN * _BLK + _BLKS_LAST * _BLK
_TAIL = _N - _TAIL0


@functools.cache
def _build_sc_call():
    mesh = plsc.VectorSubcoreMesh(core_axis_name="c", subcore_axis_name="s")
    info = plsc.get_sparse_core_info()
    nc = info.num_cores

    @functools.partial(
        pl.kernel,
        mesh=mesh,
        out_type=jax.ShapeDtypeStruct((_NW, _L), jnp.float32),
        scratch_types=[
            pltpu.VMEM((_N,), jnp.float32),
            pltpu.VMEM((_HJ, _BLK), jnp.int32),
            pltpu.VMEM((_HJ, _BLK), jnp.int32),
            pltpu.VMEM((_HJ, _BLK), jnp.float32),
            pltpu.VMEM((_HJ, _BLK), jnp.float32),
            pltpu.VMEM((_HJ, _TAIL), jnp.int32),
            pltpu.VMEM((_HJ, _TAIL), jnp.float32),
            pltpu.VMEM((_L,), jnp.float32),
            pltpu.SemaphoreType.DMA,
            pltpu.SemaphoreType.DMA,
        ],
        compiler_params=pltpu.CompilerParams(needs_layout_passes=False),
    )
    def sc_energy(q_hbm, nb_hbm, d_hbm, out_hbm,
                  qtab, nb0, nb1, d0, d1, ntb, dtb, accv, sem0, sem1):
        wid = lax.axis_index("s") * jnp.int32(nc) + lax.axis_index("c")
        astart = wid * jnp.int32(_BLKS_MAIN * _BLK)
        nblk = jnp.where(wid == jnp.int32(_NW - 1),
                         jnp.int32(_BLKS_LAST), jnp.int32(_BLKS_MAIN))
        pltpu.sync_copy(q_hbm.at[jnp.int32(0)], qtab)

        nbufs = (nb0, nb1)
        dbufs = (d0, d1)
        sems = (sem0, sem1)

        def _slices(blk, jh):
            a0 = astart + jnp.minimum(blk, nblk - 1) * jnp.int32(_BLK)
            nb_sl = nb_hbm.at[pl.ds(jh * _HJ, _HJ), pl.ds(a0, _BLK)]
            d_sl = d_hbm.at[pl.ds(jh * _HJ, _HJ), pl.ds(a0, _BLK)]
            return nb_sl, d_sl

        def start(slot, blk, jh):
            nb_sl, d_sl = _slices(blk, jh)
            pltpu.async_copy(nb_sl, nbufs[slot], sems[slot])
            pltpu.async_copy(d_sl, dbufs[slot], sems[slot])

        def wait(slot, blk, jh):
            nb_sl, d_sl = _slices(blk, jh)
            pltpu.make_async_copy(nb_sl, nbufs[slot], sems[slot]).wait()
            pltpu.make_async_copy(d_sl, dbufs[slot], sems[slot]).wait()

        def compute(slot, blk, acc):
            nbv, dv = nbufs[slot], dbufs[slot]
            a0 = astart + blk * jnp.int32(_BLK)

            def jbody(j, ss):
                out = []
                for g in range(_GRP):
                    idx = nbv[j, pl.ds(g * _L, _L)]
                    dd = dv[j, pl.ds(g * _L, _L)]
                    qj = plsc.load_gather(qtab, [idx])
                    out.append(ss[g] + qj / dd)
                return tuple(out)

            ss = lax.fori_loop(
                jnp.int32(0), jnp.int32(_HJ), jbody,
                tuple(jnp.zeros((_L,), jnp.float32) for _ in range(_GRP)),
            )
            for g in range(_GRP):
                qi = qtab[pl.ds(a0 + jnp.int32(g * _L), _L)]
                acc = acc + ss[g] * qi
            return acc

        start(0, jnp.int32(0), 0)

        def pair_body(p, acc):
            start(1, p, 1)
            wait(0, p, 0)
            acc = compute(0, p, acc)
            start(0, p + jnp.int32(1), 0)
            wait(1, p, 1)
            return compute(1, p, acc)

        acc = lax.fori_loop(
            jnp.int32(0), nblk, pair_body, jnp.zeros((_L,), jnp.float32)
        )
        wait(0, nblk - jnp.int32(1), 0)

        def tail_jbody(j, ss):
            out = []
            for g in range(_TAIL // _L):
                idx = ntb[j, pl.ds(g * _L, _L)]
                dd = dtb[j, pl.ds(g * _L, _L)]
                qj = plsc.load_gather(qtab, [idx])
                out.append(ss[g] + qj / dd)
            return tuple(out)

        t0 = jnp.int32(_TAIL0)
        tss = tuple(jnp.zeros((_L,), jnp.float32) for _ in range(_TAIL // _L))
        for jh in range(2):
            pltpu.sync_copy(
                nb_hbm.at[pl.ds(jh * _HJ, _HJ), pl.ds(t0, _TAIL)], ntb
            )
            pltpu.sync_copy(
                d_hbm.at[pl.ds(jh * _HJ, _HJ), pl.ds(t0, _TAIL)], dtb
            )
            tss = lax.fori_loop(jnp.int32(0), jnp.int32(_HJ), tail_jbody, tss)
        keep = jnp.where(wid == jnp.int32(_NW - 1),
                         jnp.float32(1.0), jnp.float32(0.0))
        for g in range(_TAIL // _L):
            qi = qtab[pl.ds(jnp.int32(_TAIL0 + g * _L), _L)]
            acc = acc + tss[g] * qi * keep

        accv[...] = acc
        pltpu.sync_copy(accv, out_hbm.at[wid])

    return sc_energy


def kernel(charges, d_ij_lr, nbmat_lr):
    nb_t = nbmat_lr.astype(jnp.int32).reshape(_N, _M).T
    d_t = d_ij_lr.reshape(_N, _M).T
    parts = _build_sc_call()(charges, nb_t, d_t)
    e = _FACTOR * jnp.sum(parts.astype(jnp.float64))
    return e.reshape(1)


# speedup vs baseline: 1.0667x; 1.0667x over previous
import functools

import jax
import jax.numpy as jnp
from jax import lax
from jax.experimental import pallas as pl
from jax.experimental.pallas import tpu as pltpu
from jax.experimental.pallas import tpu_sc as plsc

jax.config.update("jax_enable_x64", True)

_FACTOR = 13.605693122994 * 0.5291772105638411

_N = 100000
_M = 64
_NW = 32
_BLK = 128
_HJ = _M // 2
_L = 16
_GRP = _BLK // _L
_BLKS_MAIN = 25
_BLKS_LAST = 6
_TAIL0 = 31 * _BLKS_MAIN * _BLK + _BLKS_LAST * _BLK
_TAIL = _N - _TAIL0


@functools.cache
def _build_sc_call():
    mesh = plsc.VectorSubcoreMesh(core_axis_name="c", subcore_axis_name="s")
    info = plsc.get_sparse_core_info()
    nc = info.num_cores

    @functools.partial(
        pl.kernel,
        mesh=mesh,
        out_type=jax.ShapeDtypeStruct((_NW, _L), jnp.float32),
        scratch_types=[
            pltpu.VMEM((_N,), jnp.float32),
            pltpu.VMEM((_HJ, _BLK), jnp.uint32),
            pltpu.VMEM((_HJ, _BLK), jnp.uint32),
            pltpu.VMEM((_HJ, _BLK), jnp.float32),
            pltpu.VMEM((_HJ, _BLK), jnp.float32),
            pltpu.VMEM((_HJ, _TAIL), jnp.uint32),
            pltpu.VMEM((_HJ, _TAIL), jnp.float32),
            pltpu.VMEM((_L,), jnp.float32),
            pltpu.SemaphoreType.DMA,
            pltpu.SemaphoreType.DMA,
            pltpu.SemaphoreType.DMA,
        ],
        compiler_params=pltpu.CompilerParams(needs_layout_passes=False),
    )
    def sc_energy(q_hbm, nb_hbm, d_hbm, out_hbm,
                  qtab, nb0, nb1, d0, d1, ntb, dtb, accv, sem0, sem1, semq):
        wid = lax.axis_index("s") * jnp.int32(nc) + lax.axis_index("c")
        astart = wid * jnp.int32(_BLKS_MAIN * _BLK)
        nblk = jnp.where(wid == jnp.int32(_NW - 1),
                         jnp.int32(_BLKS_LAST), jnp.int32(_BLKS_MAIN))
        pltpu.async_copy(q_hbm.at[jnp.int32(0)], qtab, semq)

        nbufs = (nb0, nb1)
        dbufs = (d0, d1)
        sems = (sem0, sem1)

        def _slices(blk, jh):
            a0 = astart + jnp.minimum(blk, nblk - 1) * jnp.int32(_BLK)
            nb_sl = nb_hbm.at[pl.ds(jh * _HJ, _HJ), pl.ds(a0, _BLK)]
            d_sl = d_hbm.at[pl.ds(jh * _HJ, _HJ), pl.ds(a0, _BLK)]
            return nb_sl, d_sl

        def start(slot, blk, jh):
            nb_sl, d_sl = _slices(blk, jh)
            pltpu.async_copy(nb_sl, nbufs[slot], sems[slot])
            pltpu.async_copy(d_sl, dbufs[slot], sems[slot])

        def wait(slot, blk, jh):
            nb_sl, d_sl = _slices(blk, jh)
            pltpu.make_async_copy(nb_sl, nbufs[slot], sems[slot]).wait()
            pltpu.make_async_copy(d_sl, dbufs[slot], sems[slot]).wait()

        def compute(slot, blk, acc):
            nbv, dv = nbufs[slot], dbufs[slot]
            a0 = astart + blk * jnp.int32(_BLK)

            def jbody(j, ss):
                out = []
                for g in range(_GRP):
                    idx = plsc.bitcast(nbv[j, pl.ds(g * _L, _L)], jnp.int32)
                    dd = dv[j, pl.ds(g * _L, _L)]
                    qj = plsc.load_gather(qtab, [idx])
                    out.append(ss[g] + qj / dd)
                return tuple(out)

            ss = lax.fori_loop(
                jnp.int32(0), jnp.int32(_HJ), jbody,
                tuple(jnp.zeros((_L,), jnp.float32) for _ in range(_GRP)),
            )
            for g in range(_GRP):
                qi = qtab[pl.ds(a0 + jnp.int32(g * _L), _L)]
                acc = acc + ss[g] * qi
            return acc

        start(0, jnp.int32(0), 0)
        pltpu.make_async_copy(q_hbm.at[jnp.int32(0)], qtab, semq).wait()

        def pair_body(p, acc):
            start(1, p, 1)
            wait(0, p, 0)
            acc = compute(0, p, acc)
            start(0, p + jnp.int32(1), 0)
            wait(1, p, 1)
            return compute(1, p, acc)

        acc = lax.fori_loop(
            jnp.int32(0), nblk, pair_body, jnp.zeros((_L,), jnp.float32)
        )
        wait(0, nblk - jnp.int32(1), 0)

        def tail_jbody(j, ss):
            out = []
            for g in range(_TAIL // _L):
                idx = plsc.bitcast(ntb[j, pl.ds(g * _L, _L)], jnp.int32)
                dd = dtb[j, pl.ds(g * _L, _L)]
                qj = plsc.load_gather(qtab, [idx])
                out.append(ss[g] + qj / dd)
            return tuple(out)

        t0 = jnp.int32(_TAIL0)
        tss = tuple(jnp.zeros((_L,), jnp.float32) for _ in range(_TAIL // _L))
        for jh in range(2):
            pltpu.sync_copy(
                nb_hbm.at[pl.ds(jh * _HJ, _HJ), pl.ds(t0, _TAIL)], ntb
            )
            pltpu.sync_copy(
                d_hbm.at[pl.ds(jh * _HJ, _HJ), pl.ds(t0, _TAIL)], dtb
            )
            tss = lax.fori_loop(jnp.int32(0), jnp.int32(_HJ), tail_jbody, tss)
        keep = jnp.where(wid == jnp.int32(_NW - 1),
                         jnp.float32(1.0), jnp.float32(0.0))
        for g in range(_TAIL // _L):
            qi = qtab[pl.ds(jnp.int32(_TAIL0 + g * _L), _L)]
            acc = acc + tss[g] * qi * keep

        accv[...] = acc
        pltpu.sync_copy(accv, out_hbm.at[wid])

    return sc_energy


def kernel(charges, d_ij_lr, nbmat_lr):
    nb_t = nbmat_lr.astype(jnp.uint32).reshape(_N, _M).T
    d_t = d_ij_lr.reshape(_N, _M).T
    parts = _build_sc_call()(charges, nb_t, d_t)
    e = _FACTOR * jnp.sum(parts.astype(jnp.float64))
    return e.reshape(1)
